# Initial kernel scaffold; baseline (speedup 1.0000x reference)
import jax
import jax.numpy as jnp
from jax.experimental import pallas as pl


def kernel(x, edge_index, edge_weights, W, b):
    raise NotImplementedError("write your pallas kernel here")


# speedup vs baseline: 2.1473x; 2.1473x over previous
import functools

import jax
import jax.numpy as jnp
from jax import lax
from jax.experimental import pallas as pl
from jax.experimental.pallas import tpu as pltpu
from jax.experimental.pallas import tpu_sc as plsc

N_NODES = 10000
N_EDGES = 320000
D = 128
NC = 2
NS = 16
NHALF = N_NODES // NC
TRASH = NHALF
H_ROWS = NHALF + 8
E_PER_T = N_EDGES // NS
CHUNK = 80
N_CHUNKS = E_PER_T // CHUNK
ROWS_PER_TILE = 312
TAIL_OFF = NS * ROWS_PER_TILE
TAIL_ROWS = NHALF - TAIL_OFF


def _sc_segment(x, src, dst, ew):
    mesh = plsc.VectorSubcoreMesh(core_axis_name="c", subcore_axis_name="s")

    @functools.partial(
        pl.kernel,
        out_type=jax.ShapeDtypeStruct((NC, NHALF, D), jnp.float32),
        mesh=mesh,
        compiler_params=pltpu.CompilerParams(needs_layout_passes=False),
        scratch_types=[
            pltpu.VMEM((CHUNK,), jnp.int32),
            pltpu.VMEM((CHUNK,), jnp.int32),
            pltpu.VMEM((CHUNK,), jnp.float32),
            pltpu.VMEM((CHUNK, D), jnp.float32),
            pltpu.VMEM((ROWS_PER_TILE, D), jnp.float32),
            pltpu.VMEM_SHARED((H_ROWS, D), jnp.float32),
            pltpu.SemaphoreType.DMA,
        ],
    )
    def k(x_hbm, src_hbm, dst_hbm, ew_hbm, out_hbm,
          src_v, dst_v, ew_v, rows_v, zbuf_v, h_sh, sem):
        cid = lax.axis_index("c")
        sid = lax.axis_index("s")

        zero16 = jnp.zeros((16,), jnp.float32)

        def zrow(r, _):
            for j in range(D // 16):
                zbuf_v[r, pl.ds(j * 16, 16)] = zero16
            return 0

        lax.fori_loop(0, ROWS_PER_TILE, zrow, 0)
        pltpu.sync_copy(zbuf_v, h_sh.at[pl.ds(sid * ROWS_PER_TILE, ROWS_PER_TILE)])

        @pl.when(sid == NS - 1)
        def _zero_tail():
            pltpu.sync_copy(zbuf_v.at[pl.ds(0, TAIL_ROWS + 8)],
                            h_sh.at[pl.ds(TAIL_OFF, TAIL_ROWS + 8)])

        plsc.subcore_barrier()

        base = sid * E_PER_T
        row_lo = cid * NHALF

        def chunk_body(kk, _):
            off = base + kk * CHUNK
            pltpu.sync_copy(src_hbm.at[pl.ds(off, CHUNK)], src_v)
            pltpu.sync_copy(dst_hbm.at[pl.ds(off, CHUNK)], dst_v)
            pltpu.sync_copy(ew_hbm.at[pl.ds(off, CHUNK)], ew_v)
            pltpu.async_copy(x_hbm.at[src_v], rows_v, sem).wait()

            for g in range(CHUNK // 16):
                d16 = dst_v[pl.ds(g * 16, 16)] - row_lo
                ok = (d16 >= 0) & (d16 < NHALF)
                dst_v[pl.ds(g * 16, 16)] = jnp.where(ok, d16, TRASH)

            def edge_body(e, _):
                w16 = plsc.load_gather(ew_v, [jnp.zeros((16,), jnp.int32) + e])
                for j in range(D // 16):
                    rows_v[e, pl.ds(j * 16, 16)] = rows_v[e, pl.ds(j * 16, 16)] * w16
                return 0

            lax.fori_loop(0, CHUNK, edge_body, 0)
            pltpu.sync_copy(rows_v, h_sh.at[dst_v], add=True)
            return 0

        lax.fori_loop(0, N_CHUNKS, chunk_body, 0)
        plsc.subcore_barrier()

        pltpu.sync_copy(h_sh.at[pl.ds(sid * ROWS_PER_TILE, ROWS_PER_TILE)], zbuf_v)
        pltpu.sync_copy(zbuf_v, out_hbm.at[cid, pl.ds(sid * ROWS_PER_TILE, ROWS_PER_TILE)])

        @pl.when(sid == NS - 1)
        def _copy_tail():
            pltpu.sync_copy(h_sh.at[pl.ds(TAIL_OFF, TAIL_ROWS)],
                            rows_v.at[pl.ds(0, TAIL_ROWS)])
            pltpu.sync_copy(rows_v.at[pl.ds(0, TAIL_ROWS)],
                            out_hbm.at[cid, pl.ds(TAIL_OFF, TAIL_ROWS)])

    return k(x, src, dst, ew)


_TC_BLK = 1000


def _tc_linear(hpart, W, b2):
    def body(h_ref, w_ref, b_ref, o_ref):
        o_ref[...] = lax.dot_general(
            h_ref[0], w_ref[...], (((1,), (1,)), ((), ())),
            preferred_element_type=jnp.float32) + b_ref[...]

    nblk = NHALF // _TC_BLK

    return pl.pallas_call(
        body,
        grid=(N_NODES // _TC_BLK,),
        in_specs=[
            pl.BlockSpec((1, _TC_BLK, D), lambda i: (i // nblk, i % nblk, 0)),
            pl.BlockSpec((D, D), lambda i: (0, 0)),
            pl.BlockSpec((1, D), lambda i: (0, 0)),
        ],
        out_specs=pl.BlockSpec((_TC_BLK, D), lambda i: (i, 0)),
        out_shape=jax.ShapeDtypeStruct((N_NODES, D), jnp.float32),
    )(hpart, W, b2)


def kernel(x, edge_index, edge_weights, W, b):
    ei = edge_index.astype(jnp.int32)
    src = ei[0]
    dst = ei[1]
    ew = edge_weights.reshape(-1)
    hpart = _sc_segment(x, src, dst, ew)
    return _tc_linear(hpart, W, b.reshape(1, D))


# speedup vs baseline: 5.6269x; 2.6204x over previous
import functools

import jax
import jax.numpy as jnp
from jax import lax
from jax.experimental import pallas as pl
from jax.experimental.pallas import tpu as pltpu
from jax.experimental.pallas import tpu_sc as plsc

N_NODES = 10000
N_EDGES = 320000
D = 128
NC = 2
NS = 16
NHALF = N_NODES // NC
TRASH = NHALF
H_ROWS = NHALF + 8
E_PER_T = N_EDGES // NS
CHUNK = 80
N_CHUNKS = E_PER_T // CHUNK
N_TRIPLES = (N_CHUNKS - 4) // 3
ROWS_PER_TILE = 312
ZROWS = 104
TAIL_OFF = NS * ROWS_PER_TILE
TAIL_ROWS = NHALF - TAIL_OFF


def _sc_segment(x, src, dst, ew):
    mesh = plsc.VectorSubcoreMesh(core_axis_name="c", subcore_axis_name="s")

    @functools.partial(
        pl.kernel,
        out_type=jax.ShapeDtypeStruct((NC, NHALF, D), jnp.float32),
        mesh=mesh,
        compiler_params=pltpu.CompilerParams(needs_layout_passes=False),
        scratch_types=[
            pltpu.VMEM((CHUNK,), jnp.int32),
            pltpu.VMEM((CHUNK,), jnp.int32),
            pltpu.VMEM((CHUNK,), jnp.int32),
            pltpu.VMEM((CHUNK,), jnp.int32),
            pltpu.VMEM((CHUNK,), jnp.int32),
            pltpu.VMEM((CHUNK,), jnp.int32),
            pltpu.VMEM((CHUNK,), jnp.float32),
            pltpu.VMEM((CHUNK,), jnp.float32),
            pltpu.VMEM((CHUNK,), jnp.float32),
            pltpu.VMEM((CHUNK,), jnp.int32),
            pltpu.VMEM((CHUNK,), jnp.int32),
            pltpu.VMEM((CHUNK,), jnp.int32),
            pltpu.VMEM((CHUNK, D), jnp.float32),
            pltpu.VMEM((CHUNK, D), jnp.float32),
            pltpu.VMEM((CHUNK, D), jnp.float32),
            pltpu.VMEM((ZROWS, D), jnp.float32),
            pltpu.VMEM_SHARED((H_ROWS, D), jnp.float32),
            pltpu.SemaphoreType.DMA,
            pltpu.SemaphoreType.DMA,
            pltpu.SemaphoreType.DMA,
            pltpu.SemaphoreType.DMA,
            pltpu.SemaphoreType.DMA,
            pltpu.SemaphoreType.DMA,
            pltpu.SemaphoreType.DMA,
            pltpu.SemaphoreType.DMA,
            pltpu.SemaphoreType.DMA,
        ],
    )
    def k(x_hbm, src_hbm, dst_hbm, ew_hbm, out_hbm,
          src_c0, src_c1, src_c2, dstr_c0, dstr_c1, dstr_c2,
          ew_c0, ew_c1, ew_c2, dstc0, dstc1, dstc2,
          rows0, rows1, rows2, zbuf_v, h_sh,
          isem0, isem1, isem2, gsem0, gsem1, gsem2, ssem0, ssem1, ssem2):
        cid = lax.axis_index("c")
        sid = lax.axis_index("s")
        src_c = (src_c0, src_c1, src_c2)
        dstr_c = (dstr_c0, dstr_c1, dstr_c2)
        ew_c = (ew_c0, ew_c1, ew_c2)
        dstc = (dstc0, dstc1, dstc2)
        rows = (rows0, rows1, rows2)
        isems = (isem0, isem1, isem2)
        gsems = (gsem0, gsem1, gsem2)
        ssems = (ssem0, ssem1, ssem2)

        ebase = sid * E_PER_T
        row_lo = cid * NHALF

        def issue_idx(c, b):
            off = ebase + c * CHUNK
            pltpu.async_copy(src_hbm.at[pl.ds(off, CHUNK)], src_c[b], isems[b])
            pltpu.async_copy(dst_hbm.at[pl.ds(off, CHUNK)], dstr_c[b], isems[b])
            pltpu.async_copy(ew_hbm.at[pl.ds(off, CHUNK)], ew_c[b], isems[b])

        def wait_idx(b):
            pltpu.make_async_copy(src_hbm.at[pl.ds(0, CHUNK)], src_c[b], isems[b]).wait()
            pltpu.make_async_copy(dst_hbm.at[pl.ds(0, CHUNK)], dstr_c[b], isems[b]).wait()
            pltpu.make_async_copy(ew_hbm.at[pl.ds(0, CHUNK)], ew_c[b], isems[b]).wait()

        def issue_gather(b):
            pltpu.async_copy(x_hbm.at[src_c[b]], rows[b], gsems[b])

        def wait_gather(b):
            pltpu.make_async_copy(x_hbm.at[pl.ds(0, CHUNK)], rows[b], gsems[b]).wait()

        def issue_scatter(b):
            pltpu.async_copy(rows[b], h_sh.at[dstc[b]], ssems[b], add=True)

        def wait_scatter(b):
            pltpu.make_async_copy(rows[b], h_sh.at[pl.ds(0, CHUNK)], ssems[b]).wait()

        def process(b):
            rb = rows[b]
            db = dstc[b]
            eb = ew_c[b]
            drb = dstr_c[b]

            def grp(g, _):
                off = g * 16
                d16 = drb[pl.ds(off, 16)] - row_lo
                ok = (d16 >= 0) & (d16 < NHALF)
                db[pl.ds(off, 16)] = jnp.where(ok, d16, TRASH)
                for e2 in range(16):
                    e = off + e2
                    w16 = plsc.load_gather(eb, [jnp.full((16,), e, jnp.int32)])
                    for j in range(D // 16):
                        rb[e, pl.ds(j * 16, 16)] = rb[e, pl.ds(j * 16, 16)] * w16
                return 0

            lax.fori_loop(0, CHUNK // 16, grp, 0)

        zero16 = jnp.zeros((16,), jnp.float32)

        def zrow(r, _):
            for j in range(D // 16):
                zbuf_v[r, pl.ds(j * 16, 16)] = zero16
            return 0

        lax.fori_loop(0, ZROWS, zrow, 0)
        for kk in range(ROWS_PER_TILE // ZROWS):
            pltpu.sync_copy(zbuf_v, h_sh.at[pl.ds(sid * ROWS_PER_TILE + kk * ZROWS, ZROWS)])

        @pl.when(sid == NS - 1)
        def _zero_tail():
            pltpu.sync_copy(zbuf_v.at[pl.ds(0, TAIL_ROWS + 8)],
                            h_sh.at[pl.ds(TAIL_OFF, TAIL_ROWS + 8)])

        plsc.subcore_barrier()

        issue_idx(0, 0)
        wait_idx(0)
        issue_gather(0)
        issue_idx(1, 1)

        def triple(t, _):
            for i in range(3):
                c = 3 * t + i
                nb = (i + 1) % 3
                b2 = (i + 2) % 3
                if i < 2:
                    @pl.when(t > 0)
                    def _w():
                        wait_scatter(nb)
                else:
                    wait_scatter(nb)
                wait_idx(nb)
                issue_gather(nb)
                issue_idx(c + 2, b2)
                wait_gather(i)
                process(i)
                issue_scatter(i)
            return 0

        lax.fori_loop(0, N_TRIPLES, triple, 0)

        c0 = N_TRIPLES * 3
        for c in range(c0, N_CHUNKS):
            b = c % 3
            nb = (c + 1) % 3
            b2 = (c + 2) % 3
            if c + 1 < N_CHUNKS:
                wait_scatter(nb)
                wait_idx(nb)
                issue_gather(nb)
            if c + 2 < N_CHUNKS:
                issue_idx(c + 2, b2)
            wait_gather(b)
            process(b)
            issue_scatter(b)
        for b in range(3):
            wait_scatter(b)
        plsc.subcore_barrier()

        for kk in range(ROWS_PER_TILE // ZROWS):
            off = sid * ROWS_PER_TILE + kk * ZROWS
            pltpu.sync_copy(h_sh.at[pl.ds(off, ZROWS)], zbuf_v)
            pltpu.sync_copy(zbuf_v, out_hbm.at[cid, pl.ds(off, ZROWS)])

        @pl.when(sid == NS - 1)
        def _copy_tail():
            pltpu.sync_copy(h_sh.at[pl.ds(TAIL_OFF, TAIL_ROWS)],
                            rows0.at[pl.ds(0, TAIL_ROWS)])
            pltpu.sync_copy(rows0.at[pl.ds(0, TAIL_ROWS)],
                            out_hbm.at[cid, pl.ds(TAIL_OFF, TAIL_ROWS)])

    return k(x, src, dst, ew)


_TC_BLK = 1000


def _tc_linear(hpart, W, b2):
    def body(h_ref, w_ref, b_ref, o_ref):
        o_ref[...] = lax.dot_general(
            h_ref[0], w_ref[...], (((1,), (1,)), ((), ())),
            preferred_element_type=jnp.float32) + b_ref[...]

    nblk = NHALF // _TC_BLK

    return pl.pallas_call(
        body,
        grid=(N_NODES // _TC_BLK,),
        in_specs=[
            pl.BlockSpec((1, _TC_BLK, D), lambda i: (i // nblk, i % nblk, 0)),
            pl.BlockSpec((D, D), lambda i: (0, 0)),
            pl.BlockSpec((1, D), lambda i: (0, 0)),
        ],
        out_specs=pl.BlockSpec((_TC_BLK, D), lambda i: (i, 0)),
        out_shape=jax.ShapeDtypeStruct((N_NODES, D), jnp.float32),
    )(hpart, W, b2)


def kernel(x, edge_index, edge_weights, W, b):
    ei = edge_index.astype(jnp.int32)
    src = ei[0]
    dst = ei[1]
    ew = edge_weights.reshape(-1)
    hpart = _sc_segment(x, src, dst, ew)
    return _tc_linear(hpart, W, b.reshape(1, D))


# speedup vs baseline: 5.8805x; 1.0451x over previous
import functools

import jax
import jax.numpy as jnp
from jax import lax
from jax.experimental import pallas as pl
from jax.experimental.pallas import tpu as pltpu
from jax.experimental.pallas import tpu_sc as plsc

N_NODES = 10000
N_EDGES = 320000
D = 128
NC = 2
NS = 16
NHALF = N_NODES // NC
TRASH = NHALF
H_ROWS = NHALF + 8
E_PER_T = N_EDGES // NS
CHUNK = 80
N_CHUNKS = E_PER_T // CHUNK
N_TRIPLES = (N_CHUNKS - 4) // 3
ROWS_PER_TILE = 312
ZROWS = 104
TAIL_OFF = NS * ROWS_PER_TILE
TAIL_ROWS = NHALF - TAIL_OFF


def _sc_segment(x, src, dst, ew):
    mesh = plsc.VectorSubcoreMesh(core_axis_name="c", subcore_axis_name="s")

    @functools.partial(
        pl.kernel,
        out_type=jax.ShapeDtypeStruct((NC, NHALF, D), jnp.float32),
        mesh=mesh,
        compiler_params=pltpu.CompilerParams(needs_layout_passes=False),
        scratch_types=[
            pltpu.VMEM((CHUNK,), jnp.int32),
            pltpu.VMEM((CHUNK,), jnp.int32),
            pltpu.VMEM((CHUNK,), jnp.int32),
            pltpu.VMEM((CHUNK,), jnp.int32),
            pltpu.VMEM((CHUNK,), jnp.int32),
            pltpu.VMEM((CHUNK,), jnp.int32),
            pltpu.VMEM((CHUNK,), jnp.float32),
            pltpu.VMEM((CHUNK,), jnp.float32),
            pltpu.VMEM((CHUNK,), jnp.float32),
            pltpu.VMEM((CHUNK,), jnp.int32),
            pltpu.VMEM((CHUNK,), jnp.int32),
            pltpu.VMEM((CHUNK,), jnp.int32),
            pltpu.VMEM((CHUNK, D), jnp.float32),
            pltpu.VMEM((CHUNK, D), jnp.float32),
            pltpu.VMEM((CHUNK, D), jnp.float32),
            pltpu.VMEM((ZROWS, D), jnp.float32),
            pltpu.VMEM_SHARED((H_ROWS, D), jnp.float32),
            pltpu.SemaphoreType.DMA,
            pltpu.SemaphoreType.DMA,
            pltpu.SemaphoreType.DMA,
            pltpu.SemaphoreType.DMA,
            pltpu.SemaphoreType.DMA,
            pltpu.SemaphoreType.DMA,
            pltpu.SemaphoreType.DMA,
            pltpu.SemaphoreType.DMA,
            pltpu.SemaphoreType.DMA,
        ],
    )
    def k(x_hbm, src_hbm, dst_hbm, ew_hbm, out_hbm,
          src_c0, src_c1, src_c2, dstr_c0, dstr_c1, dstr_c2,
          ew_c0, ew_c1, ew_c2, dstc0, dstc1, dstc2,
          rows0, rows1, rows2, zbuf_v, h_sh,
          isem0, isem1, isem2, gsem0, gsem1, gsem2, ssem0, ssem1, ssem2):
        cid = lax.axis_index("c")
        sid = lax.axis_index("s")
        src_c = (src_c0, src_c1, src_c2)
        dstr_c = (dstr_c0, dstr_c1, dstr_c2)
        ew_c = (ew_c0, ew_c1, ew_c2)
        dstc = (dstc0, dstc1, dstc2)
        rows = (rows0, rows1, rows2)
        isems = (isem0, isem1, isem2)
        gsems = (gsem0, gsem1, gsem2)
        ssems = (ssem0, ssem1, ssem2)

        ebase = sid * E_PER_T
        row_lo = cid * NHALF

        def issue_idx(c, b):
            off = ebase + c * CHUNK
            pltpu.async_copy(src_hbm.at[pl.ds(off, CHUNK)], src_c[b], isems[b])
            pltpu.async_copy(dst_hbm.at[pl.ds(off, CHUNK)], dstr_c[b], isems[b])
            pltpu.async_copy(ew_hbm.at[pl.ds(off, CHUNK)], ew_c[b], isems[b])

        def wait_idx(b):
            pltpu.make_async_copy(src_hbm.at[pl.ds(0, CHUNK)], src_c[b], isems[b]).wait()
            pltpu.make_async_copy(dst_hbm.at[pl.ds(0, CHUNK)], dstr_c[b], isems[b]).wait()
            pltpu.make_async_copy(ew_hbm.at[pl.ds(0, CHUNK)], ew_c[b], isems[b]).wait()

        def issue_gather(b):
            pltpu.async_copy(x_hbm.at[src_c[b]], rows[b], gsems[b])

        def wait_gather(b):
            pltpu.make_async_copy(x_hbm.at[pl.ds(0, CHUNK)], rows[b], gsems[b]).wait()

        def issue_scatter(b):
            pltpu.async_copy(rows[b], h_sh.at[dstc[b]], ssems[b], add=True)

        def wait_scatter(b):
            pltpu.make_async_copy(rows[b], h_sh.at[pl.ds(0, CHUNK)], ssems[b]).wait()

        def process(b):
            rb = rows[b]
            db = dstc[b]
            eb = ew_c[b]
            drb = dstr_c[b]

            @plsc.parallel_loop(0, CHUNK // 16)
            def grp(g):
                off = g * 16
                d16 = drb[pl.ds(off, 16)] - row_lo
                ok = (d16 >= 0) & (d16 < NHALF)
                db[pl.ds(off, 16)] = jnp.where(ok, d16, TRASH)
                w16 = eb[pl.ds(off, 16)]
                for e2 in range(16):
                    e = off + e2
                    wb = jnp.full((16,), w16[e2])
                    for j in range(D // 16):
                        rb[e, pl.ds(j * 16, 16)] = rb[e, pl.ds(j * 16, 16)] * wb

        zero16 = jnp.zeros((16,), jnp.float32)

        def zrow(r, _):
            for j in range(D // 16):
                zbuf_v[r, pl.ds(j * 16, 16)] = zero16
            return 0

        lax.fori_loop(0, ZROWS, zrow, 0)
        for kk in range(ROWS_PER_TILE // ZROWS):
            pltpu.sync_copy(zbuf_v, h_sh.at[pl.ds(sid * ROWS_PER_TILE + kk * ZROWS, ZROWS)])

        @pl.when(sid == NS - 1)
        def _zero_tail():
            pltpu.sync_copy(zbuf_v.at[pl.ds(0, TAIL_ROWS + 8)],
                            h_sh.at[pl.ds(TAIL_OFF, TAIL_ROWS + 8)])

        plsc.subcore_barrier()

        issue_idx(0, 0)
        wait_idx(0)
        issue_gather(0)
        issue_idx(1, 1)

        def triple(t, _):
            for i in range(3):
                c = 3 * t + i
                nb = (i + 1) % 3
                b2 = (i + 2) % 3
                if i < 2:
                    @pl.when(t > 0)
                    def _w():
                        wait_scatter(nb)
                else:
                    wait_scatter(nb)
                wait_idx(nb)
                issue_gather(nb)
                issue_idx(c + 2, b2)
                wait_gather(i)
                process(i)
                issue_scatter(i)
            return 0

        lax.fori_loop(0, N_TRIPLES, triple, 0)

        c0 = N_TRIPLES * 3
        for c in range(c0, N_CHUNKS):
            b = c % 3
            nb = (c + 1) % 3
            b2 = (c + 2) % 3
            if c + 1 < N_CHUNKS:
                wait_scatter(nb)
                wait_idx(nb)
                issue_gather(nb)
            if c + 2 < N_CHUNKS:
                issue_idx(c + 2, b2)
            wait_gather(b)
            process(b)
            issue_scatter(b)
        for b in range(3):
            wait_scatter(b)
        plsc.subcore_barrier()

        for kk in range(ROWS_PER_TILE // ZROWS):
            off = sid * ROWS_PER_TILE + kk * ZROWS
            pltpu.sync_copy(h_sh.at[pl.ds(off, ZROWS)], zbuf_v)
            pltpu.sync_copy(zbuf_v, out_hbm.at[cid, pl.ds(off, ZROWS)])

        @pl.when(sid == NS - 1)
        def _copy_tail():
            pltpu.sync_copy(h_sh.at[pl.ds(TAIL_OFF, TAIL_ROWS)],
                            rows0.at[pl.ds(0, TAIL_ROWS)])
            pltpu.sync_copy(rows0.at[pl.ds(0, TAIL_ROWS)],
                            out_hbm.at[cid, pl.ds(TAIL_OFF, TAIL_ROWS)])

    return k(x, src, dst, ew)


_TC_BLK = 1000


def _tc_linear(hpart, W, b2):
    def body(h_ref, w_ref, b_ref, o_ref):
        o_ref[...] = lax.dot_general(
            h_ref[0], w_ref[...], (((1,), (1,)), ((), ())),
            preferred_element_type=jnp.float32) + b_ref[...]

    nblk = NHALF // _TC_BLK

    return pl.pallas_call(
        body,
        grid=(N_NODES // _TC_BLK,),
        in_specs=[
            pl.BlockSpec((1, _TC_BLK, D), lambda i: (i // nblk, i % nblk, 0)),
            pl.BlockSpec((D, D), lambda i: (0, 0)),
            pl.BlockSpec((1, D), lambda i: (0, 0)),
        ],
        out_specs=pl.BlockSpec((_TC_BLK, D), lambda i: (i, 0)),
        out_shape=jax.ShapeDtypeStruct((N_NODES, D), jnp.float32),
    )(hpart, W, b2)


def kernel(x, edge_index, edge_weights, W, b):
    ei = edge_index.astype(jnp.int32)
    src = ei[0]
    dst = ei[1]
    ew = edge_weights.reshape(-1)
    hpart = _sc_segment(x, src, dst, ew)
    return _tc_linear(hpart, W, b.reshape(1, D))
